# baseline (speedup 1.0000x reference)
import jax
import jax.numpy as jnp
from jax.experimental import pallas as pl

N = 50000
H1, H2, C = 4, 2, 64


def _gat_conv(x, edge_index, edge_attr, W, a_src, a_dst, We, ae, b, heads, out_ch):
    n = x.shape[0]
    loop = jnp.arange(n, dtype=edge_index.dtype)
    ei = jnp.concatenate([edge_index, jnp.stack([loop, loop])], axis=1)
    mean_attr = jnp.mean(edge_attr, axis=0, keepdims=True)
    ea = jnp.concatenate([edge_attr, jnp.broadcast_to(mean_attr, (n, edge_attr.shape[1]))], axis=0)
    src, dst = ei[0], ei[1]
    h = (x @ W).reshape(n, heads, out_ch)
    alpha_src = jnp.sum(h * a_src[None], axis=-1)
    alpha_dst = jnp.sum(h * a_dst[None], axis=-1)
    efeat = (ea @ We).reshape(-1, heads, out_ch)
    alpha_edge = jnp.sum(efeat * ae[None], axis=-1)
    alpha = alpha_src[src] + alpha_dst[dst] + alpha_edge
    alpha = jax.nn.leaky_relu(alpha, negative_slope=0.2)
    amax = jax.ops.segment_max(alpha, dst, num_segments=n)
    alpha = jnp.exp(alpha - amax[dst])
    denom = jax.ops.segment_sum(alpha, dst, num_segments=n)
    alpha = alpha / (denom[dst] + 1e-16)
    out = jax.ops.segment_sum(h[src] * alpha[:, :, None], dst, num_segments=n)
    return out.reshape(n, heads * out_ch) + b


def _graph_norm(x, w, b, ms, eps=1e-5):
    mean = jnp.mean(x, axis=0, keepdims=True)
    out = x - ms * mean
    var = jnp.mean(out * out, axis=0, keepdims=True)
    return w * out / jnp.sqrt(var + eps) + b


def _cls_kernel(h_ref, w_ref, b_ref, o_ref):
    o_ref[...] = jnp.dot(h_ref[...], w_ref[...], preferred_element_type=jnp.float32) + b_ref[...]


def _classifier(h, Wc, bc):
    n, f = h.shape
    ncls = Wc.shape[1]
    blk = 2000
    return pl.pallas_call(
        _cls_kernel,
        grid=(n // blk,),
        in_specs=[
            pl.BlockSpec((blk, f), lambda i: (i, 0)),
            pl.BlockSpec((f, ncls), lambda i: (0, 0)),
            pl.BlockSpec((ncls,), lambda i: (0,)),
        ],
        out_specs=pl.BlockSpec((blk, ncls), lambda i: (i, 0)),
        out_shape=jax.ShapeDtypeStruct((n, ncls), jnp.float32),
    )(h, Wc, bc)


def kernel(x, edge_index, edge_attr, W1, a_src1, a_dst1, We1, ae1, b1, gnw1, gnb1, gnm1, W2, a_src2, a_dst2, We2, ae2, b2, gnw2, gnb2, gnm2, Wc, bc):
    h = _gat_conv(x, edge_index, edge_attr, W1, a_src1, a_dst1, We1, ae1, b1, H1, C)
    h = _graph_norm(h, gnw1, gnb1, gnm1)
    h = jax.nn.elu(h)
    h = _gat_conv(h, edge_index, edge_attr, W2, a_src2, a_dst2, We2, ae2, b2, H2, C)
    h = _graph_norm(h, gnw2, gnb2, gnm2)
    h = jax.nn.elu(h)
    return _classifier(h, Wc, bc)


# speedup vs baseline: 10.0558x; 10.0558x over previous
import functools

import numpy as np
import jax
import jax.numpy as jnp
from jax import lax
from jax.experimental import pallas as pl
from jax.experimental.pallas import tpu as pltpu
from jax.experimental.pallas import tpu_sc as plsc

N = 50000
E = 800000
NPAD = 50176
EP = 802816
NTILE = 32
STR = NPAD // 16
BLK = 3136
NBLK = NPAD // BLK


def _dense_head_body(x_ref, w_ref, ps_ref, pd_ref,
                     h_ref, as_ref, ad_ref, ms_ref, md_ref):
    i = pl.program_id(0)
    hb = jnp.dot(x_ref[...].astype(jnp.bfloat16), w_ref[...].astype(jnp.bfloat16),
                 preferred_element_type=jnp.float32)
    h_ref[...] = hb
    asb = jnp.dot(hb, ps_ref[...], preferred_element_type=jnp.float32,
                  precision=lax.Precision.HIGHEST)
    adb = jnp.dot(hb, pd_ref[...], preferred_element_type=jnp.float32,
                  precision=lax.Precision.HIGHEST)
    as_ref[...] = asb
    ad_ref[...] = adb

    @pl.when(i == 0)
    def _():
        ms_ref[...] = jnp.full_like(ms_ref, -1e30)
        md_ref[...] = jnp.full_like(md_ref, -1e30)

    ms_ref[...] = jnp.maximum(ms_ref[...], jnp.max(asb, axis=0, keepdims=True))
    md_ref[...] = jnp.maximum(md_ref[...], jnp.max(adb, axis=0, keepdims=True))


def _dense_head(xp, W, Psrc, Pdst, H):
    fin = xp.shape[1]
    fout = W.shape[1]
    return pl.pallas_call(
        _dense_head_body,
        grid=(NBLK,),
        in_specs=[
            pl.BlockSpec((BLK, fin), lambda i: (i, 0)),
            pl.BlockSpec((fin, fout), lambda i: (0, 0)),
            pl.BlockSpec((fout, H), lambda i: (0, 0)),
            pl.BlockSpec((fout, H), lambda i: (0, 0)),
        ],
        out_specs=[
            pl.BlockSpec((BLK, fout), lambda i: (i, 0)),
            pl.BlockSpec((BLK, H), lambda i: (i, 0)),
            pl.BlockSpec((BLK, H), lambda i: (i, 0)),
            pl.BlockSpec((1, H), lambda i: (0, 0)),
            pl.BlockSpec((1, H), lambda i: (0, 0)),
        ],
        out_shape=[
            jax.ShapeDtypeStruct((NPAD, fout), jnp.float32),
            jax.ShapeDtypeStruct((NPAD, H), jnp.float32),
            jax.ShapeDtypeStruct((NPAD, H), jnp.float32),
            jax.ShapeDtypeStruct((1, H), jnp.float32),
            jax.ShapeDtypeStruct((1, H), jnp.float32),
        ],
    )(xp, W, Psrc, Pdst)


def _edge_logits_body(ea_ref, we_ref, pae_ref,
                      aed_ref, sum_ref, mx_ref, loop_ref, *, nblocks):
    i = pl.program_id(0)
    we16 = we_ref[...].astype(jnp.bfloat16).astype(jnp.float32)
    weeff = jnp.dot(we16, pae_ref[...], preferred_element_type=jnp.float32,
                    precision=lax.Precision.HIGHEST)
    eab = ea_ref[...]
    eab16 = eab.astype(jnp.bfloat16).astype(jnp.float32)
    aeb = jnp.dot(eab16, weeff, preferred_element_type=jnp.float32,
                  precision=lax.Precision.HIGHEST)
    aed_ref[...] = aeb

    @pl.when(i == 0)
    def _():
        sum_ref[...] = jnp.zeros_like(sum_ref)
        mx_ref[...] = jnp.full_like(mx_ref, -1e30)

    sum_ref[...] += jnp.sum(eab, axis=0, keepdims=True)
    mx_ref[...] = jnp.maximum(mx_ref[...], jnp.max(aeb, axis=0, keepdims=True))

    @pl.when(i == nblocks - 1)
    def _():
        mean16 = (sum_ref[...] / E).astype(jnp.bfloat16).astype(jnp.float32)
        loop_ref[...] = jnp.dot(mean16, weeff,
                                preferred_element_type=jnp.float32,
                                precision=lax.Precision.HIGHEST)


def _edge_logits(edge_attr, We, Pae, H):
    fe = edge_attr.shape[1]
    blke = 8000
    nblocks = E // blke
    return pl.pallas_call(
        functools.partial(_edge_logits_body, nblocks=nblocks),
        grid=(nblocks,),
        in_specs=[
            pl.BlockSpec((blke, fe), lambda i: (i, 0)),
            pl.BlockSpec((fe, H * 64), lambda i: (0, 0)),
            pl.BlockSpec((H * 64, H), lambda i: (0, 0)),
        ],
        out_specs=[
            pl.BlockSpec((blke, H), lambda i: (i, 0)),
            pl.BlockSpec((1, fe), lambda i: (0, 0)),
            pl.BlockSpec((1, H), lambda i: (0, 0)),
            pl.BlockSpec((1, H), lambda i: (0, 0)),
        ],
        out_shape=[
            jax.ShapeDtypeStruct((E, H), jnp.float32),
            jax.ShapeDtypeStruct((1, fe), jnp.float32),
            jax.ShapeDtypeStruct((1, H), jnp.float32),
            jax.ShapeDtypeStruct((1, H), jnp.float32),
        ],
    )(edge_attr, We, Pae)


def _epilogue_body(p0_ref, p1_ref, d0_ref, d1_ref, h_ref, as_ref, ad_ref,
                   loop_ref, m_ref, q_ref, b_ref,
                   out_ref, cs_ref, cs2_ref, *, H, blk):
    i = pl.program_id(0)
    av = as_ref[...] + ad_ref[...] + loop_ref[...]
    av = jnp.where(av > 0, av, av * 0.2)
    el = jnp.exp(av - m_ref[...])
    den = d0_ref[...][:, :H] + d1_ref[...][:, :H] + el
    el_b = jnp.dot(el, q_ref[...], preferred_element_type=jnp.float32,
                   precision=lax.Precision.HIGHEST)
    den_b = jnp.dot(den, q_ref[...], preferred_element_type=jnp.float32,
                    precision=lax.Precision.HIGHEST)
    num = p0_ref[...] + p1_ref[...] + el_b * h_ref[...]
    op = num / den_b + b_ref[...]
    rid = lax.broadcasted_iota(jnp.int32, (blk, 1), 0) + i * blk
    op = jnp.where(rid < N, op, 0.0)
    out_ref[...] = op

    @pl.when(i == 0)
    def _():
        cs_ref[...] = jnp.zeros_like(cs_ref)
        cs2_ref[...] = jnp.zeros_like(cs2_ref)

    cs_ref[...] += jnp.sum(op, axis=0, keepdims=True)
    cs2_ref[...] += jnp.sum(op * op, axis=0, keepdims=True)


def _epilogue(p0, p1, d0, d1, hmat, as_, ad_, aeloop, M, Q, bvec, H, F):
    return pl.pallas_call(
        functools.partial(_epilogue_body, H=H, blk=BLK),
        grid=(NBLK,),
        in_specs=[
            pl.BlockSpec((BLK, F), lambda i: (i, 0)),
            pl.BlockSpec((BLK, F), lambda i: (i, 0)),
            pl.BlockSpec((BLK, 32), lambda i: (i, 0)),
            pl.BlockSpec((BLK, 32), lambda i: (i, 0)),
            pl.BlockSpec((BLK, F), lambda i: (i, 0)),
            pl.BlockSpec((BLK, H), lambda i: (i, 0)),
            pl.BlockSpec((BLK, H), lambda i: (i, 0)),
            pl.BlockSpec((1, H), lambda i: (0, 0)),
            pl.BlockSpec((1, H), lambda i: (0, 0)),
            pl.BlockSpec((H, F), lambda i: (0, 0)),
            pl.BlockSpec((1, F), lambda i: (0, 0)),
        ],
        out_specs=[
            pl.BlockSpec((BLK, F), lambda i: (i, 0)),
            pl.BlockSpec((1, F), lambda i: (0, 0)),
            pl.BlockSpec((1, F), lambda i: (0, 0)),
        ],
        out_shape=[
            jax.ShapeDtypeStruct((NPAD, F), jnp.float32),
            jax.ShapeDtypeStruct((1, F), jnp.float32),
            jax.ShapeDtypeStruct((1, F), jnp.float32),
        ],
    )(p0, p1, d0, d1, hmat, as_, ad_, aeloop, M, Q, bvec)


def _transform_body(op_ref, cs_ref, cs2_ref, gw_ref, gb_ref, gm_ref,
                    w_ref, ps_ref, pd_ref,
                    h_ref, as_ref, ad_ref, ms_ref, md_ref):
    i = pl.program_id(0)
    mean = cs_ref[...] / N
    ex2 = cs2_ref[...] / N
    msv = gm_ref[...]
    var = ex2 - msv * (2.0 - msv) * mean * mean
    scale = gw_ref[...] / jnp.sqrt(var + 1e-5)
    xb = (op_ref[...] - msv * mean) * scale + gb_ref[...]
    xb = jnp.where(xb > 0, xb, jnp.exp(xb) - 1.0)
    hb = jnp.dot(xb.astype(jnp.bfloat16), w_ref[...].astype(jnp.bfloat16),
                 preferred_element_type=jnp.float32)
    h_ref[...] = hb
    asb = jnp.dot(hb, ps_ref[...], preferred_element_type=jnp.float32,
                  precision=lax.Precision.HIGHEST)
    adb = jnp.dot(hb, pd_ref[...], preferred_element_type=jnp.float32,
                  precision=lax.Precision.HIGHEST)
    as_ref[...] = asb
    ad_ref[...] = adb

    @pl.when(i == 0)
    def _():
        ms_ref[...] = jnp.full_like(ms_ref, -1e30)
        md_ref[...] = jnp.full_like(md_ref, -1e30)

    ms_ref[...] = jnp.maximum(ms_ref[...], jnp.max(asb, axis=0, keepdims=True))
    md_ref[...] = jnp.maximum(md_ref[...], jnp.max(adb, axis=0, keepdims=True))


def _transform(op, cs, cs2, gnw, gnb, gnm, W, Psrc, Pdst, H2):
    F = op.shape[1]
    F2 = W.shape[1]
    return pl.pallas_call(
        _transform_body,
        grid=(NBLK,),
        in_specs=[
            pl.BlockSpec((BLK, F), lambda i: (i, 0)),
            pl.BlockSpec((1, F), lambda i: (0, 0)),
            pl.BlockSpec((1, F), lambda i: (0, 0)),
            pl.BlockSpec((1, F), lambda i: (0, 0)),
            pl.BlockSpec((1, F), lambda i: (0, 0)),
            pl.BlockSpec((1, F), lambda i: (0, 0)),
            pl.BlockSpec((F, F2), lambda i: (0, 0)),
            pl.BlockSpec((F2, H2), lambda i: (0, 0)),
            pl.BlockSpec((F2, H2), lambda i: (0, 0)),
        ],
        out_specs=[
            pl.BlockSpec((BLK, F2), lambda i: (i, 0)),
            pl.BlockSpec((BLK, H2), lambda i: (i, 0)),
            pl.BlockSpec((BLK, H2), lambda i: (i, 0)),
            pl.BlockSpec((1, H2), lambda i: (0, 0)),
            pl.BlockSpec((1, H2), lambda i: (0, 0)),
        ],
        out_shape=[
            jax.ShapeDtypeStruct((NPAD, F2), jnp.float32),
            jax.ShapeDtypeStruct((NPAD, H2), jnp.float32),
            jax.ShapeDtypeStruct((NPAD, H2), jnp.float32),
            jax.ShapeDtypeStruct((1, H2), jnp.float32),
            jax.ShapeDtypeStruct((1, H2), jnp.float32),
        ],
    )(op, cs, cs2, gnw, gnb, gnm, W, Psrc, Pdst)


def _classifier_body(op_ref, cs_ref, cs2_ref, gw_ref, gb_ref, gm_ref,
                     wc_ref, bc_ref, out_ref):
    mean = cs_ref[...] / N
    ex2 = cs2_ref[...] / N
    msv = gm_ref[...]
    var = ex2 - msv * (2.0 - msv) * mean * mean
    scale = gw_ref[...] / jnp.sqrt(var + 1e-5)
    xb = (op_ref[...] - msv * mean) * scale + gb_ref[...]
    xb = jnp.where(xb > 0, xb, jnp.exp(xb) - 1.0)
    out_ref[...] = jnp.dot(xb.astype(jnp.bfloat16), wc_ref[...].astype(jnp.bfloat16),
                           preferred_element_type=jnp.float32) + bc_ref[...]


def _classifier(op, cs, cs2, gnw, gnb, gnm, Wc, bc):
    F = op.shape[1]
    ncls = Wc.shape[1]
    return pl.pallas_call(
        _classifier_body,
        grid=(NBLK,),
        in_specs=[
            pl.BlockSpec((BLK, F), lambda i: (i, 0)),
            pl.BlockSpec((1, F), lambda i: (0, 0)),
            pl.BlockSpec((1, F), lambda i: (0, 0)),
            pl.BlockSpec((1, F), lambda i: (0, 0)),
            pl.BlockSpec((1, F), lambda i: (0, 0)),
            pl.BlockSpec((1, F), lambda i: (0, 0)),
            pl.BlockSpec((F, ncls), lambda i: (0, 0)),
            pl.BlockSpec((1, ncls), lambda i: (0, 0)),
        ],
        out_specs=pl.BlockSpec((BLK, ncls), lambda i: (i, 0)),
        out_shape=jax.ShapeDtypeStruct((NPAD, ncls), jnp.float32),
    )(op, cs, cs2, gnw, gnb, gnm, Wc, bc)


_MESH = dict(core_axis_name="c", subcore_axis_name="s")


def _make_pass_a(H):
    ept = EP // NTILE
    nb = ept // 512

    @functools.partial(
        pl.kernel,
        out_type=jax.ShapeDtypeStruct((H, EP), jnp.float32),
        mesh=plsc.VectorSubcoreMesh(**_MESH),
        compiler_params=pltpu.CompilerParams(needs_layout_passes=False, use_tc_tiling_on_sc=False),
        scratch_types=[
            pltpu.VMEM((NPAD,), jnp.float32),
            pltpu.VMEM((NPAD,), jnp.float32),
            pltpu.VMEM((512,), jnp.int32),
            pltpu.VMEM((512,), jnp.int32),
            pltpu.VMEM((512,), jnp.float32),
            pltpu.VMEM((512,), jnp.float32),
            pltpu.VMEM((16,), jnp.float32),
        ],
    )
    def k(asT, adT, aeT, src, dst, mvecs, esc_out, tA, tB, sbuf, dbuf, aebuf, ebuf, mv):
        c = lax.axis_index("c")
        s = lax.axis_index("s")
        base = (c * 16 + s) * ept
        for h in range(H):
            pltpu.sync_copy(asT.at[h], tA)
            pltpu.sync_copy(adT.at[h], tB)
            pltpu.sync_copy(mvecs.at[h], mv)
            mvv = mv[...]

            def body(b, carry):
                off = base + b * 512
                pltpu.sync_copy(src.at[pl.ds(off, 512)], sbuf)
                pltpu.sync_copy(dst.at[pl.ds(off, 512)], dbuf)
                pltpu.sync_copy(aeT.at[h, pl.ds(off, 512)], aebuf)

                def sub(j, carry2):
                    si = sbuf[pl.ds(j * 16, 16)]
                    di = dbuf[pl.ds(j * 16, 16)]
                    a = (plsc.load_gather(tA, [si])
                         + plsc.load_gather(tB, [di])
                         + aebuf[pl.ds(j * 16, 16)])
                    a = jnp.where(a > 0, a, a * 0.2)
                    ev = jnp.exp(a - mvv)
                    gidx = off + j * 16 + lax.iota(jnp.int32, 16)
                    ev = jnp.where(gidx < E, ev, 0.0)
                    ebuf[pl.ds(j * 16, 16)] = ev
                    return carry2

                lax.fori_loop(0, 32, sub, 0)
                pltpu.sync_copy(ebuf, esc_out.at[h, pl.ds(off, 512)])
                return carry

            lax.fori_loop(0, nb, body, 0)

    return k


def _make_stage2(G, H, F):
    epc = EP // 2
    ept = epc // 16
    nbt = ept // 128

    @functools.partial(
        pl.kernel,
        out_type=(
            jax.ShapeDtypeStruct((2, NPAD, F), jnp.float32),
            jax.ShapeDtypeStruct((2, NPAD, 32), jnp.float32),
        ),
        mesh=plsc.VectorSubcoreMesh(**_MESH),
        compiler_params=pltpu.CompilerParams(needs_layout_passes=False, use_tc_tiling_on_sc=False),
        scratch_types=[
            pltpu.VMEM_SHARED((NPAD, 32), jnp.float32),
            pltpu.VMEM((128,), jnp.int32),
            pltpu.VMEM((128,), jnp.int32),
            pltpu.VMEM((128,), jnp.float32),
            pltpu.VMEM((128, 32), jnp.float32),
            pltpu.SemaphoreType.DMA,
        ],
    )
    def k(src, dst, escT, zrows, *rest):
        hgs = rest[:G]
        pout, dout = rest[G], rest[G + 1]
        acc, idxb, dstb, escb, rowb, sem = rest[G + 2:]
        c = lax.axis_index("c")
        s = lax.axis_index("s")
        base = c * epc + s * ept
        roff = s * STR
        iota16 = lax.iota(jnp.int32, 16)

        for g in range(G):
            h = g // 2
            pltpu.sync_copy(zrows, acc.at[pl.ds(roff, STR)])
            plsc.subcore_barrier()

            def body(b, carry, _g=g, _h=h):
                off = base + b * 128
                pltpu.sync_copy(src.at[pl.ds(off, 128)], idxb)
                pltpu.sync_copy(dst.at[pl.ds(off, 128)], dstb)
                pltpu.sync_copy(escT.at[_h, pl.ds(off, 128)], escb)
                pltpu.async_copy(hgs[_g].at[idxb], rowb, sem).wait()

                def sub(j, carry2):
                    ridx = j * 16 + iota16
                    ev = escb[pl.ds(j * 16, 16)]
                    for col in range(32):
                        ci = jnp.full((16,), col, jnp.int32)
                        v = plsc.load_gather(rowb, [ridx, ci])
                        plsc.store_scatter(rowb, [ridx, ci], v * ev)
                    return carry2

                lax.fori_loop(0, 8, sub, 0)
                pltpu.sync_copy(rowb, acc.at[dstb], add=True)
                return carry

            lax.fori_loop(0, nbt, body, 0)
            plsc.subcore_barrier()
            pltpu.sync_copy(acc.at[pl.ds(roff, STR)],
                            pout.at[c, pl.ds(roff, STR), pl.ds(g * 32, 32)])
            pltpu.sync_copy(zrows, acc.at[pl.ds(roff, STR)])
            plsc.subcore_barrier()

        def zsub(j, carry2):
            ridx = j * 16 + iota16
            z16 = jnp.zeros((16,), jnp.float32)
            for col in range(32):
                ci = jnp.full((16,), col, jnp.int32)
                plsc.store_scatter(rowb, [ridx, ci], z16)
            return carry2

        lax.fori_loop(0, 8, zsub, 0)

        def dbody(b, carry):
            off = base + b * 128
            pltpu.sync_copy(dst.at[pl.ds(off, 128)], dstb)
            for h in range(H):
                pltpu.sync_copy(escT.at[h, pl.ds(off, 128)], escb)

                def dsub(j, carry2, _h=h):
                    ridx = j * 16 + iota16
                    ev = escb[pl.ds(j * 16, 16)]
                    ci = jnp.full((16,), _h, jnp.int32)
                    plsc.store_scatter(rowb, [ridx, ci], ev)
                    return carry2

                lax.fori_loop(0, 8, dsub, 0)
            pltpu.sync_copy(rowb, acc.at[dstb], add=True)
            return carry

        lax.fori_loop(0, nbt, dbody, 0)
        plsc.subcore_barrier()
        pltpu.sync_copy(acc.at[pl.ds(roff, STR)], dout.at[c, pl.ds(roff, STR)])

    return k


_PASS_A = {}
_STAGE2 = {}


def _get_pass_a(H):
    if H not in _PASS_A:
        _PASS_A[H] = _make_pass_a(H)
    return _PASS_A[H]


def _get_stage2(G, H, F):
    key = (G, H, F)
    if key not in _STAGE2:
        _STAGE2[key] = _make_stage2(G, H, F)
    return _STAGE2[key]


def _head_proj(a, F):
    H, C = a.shape
    rows = jnp.arange(F)
    cols = rows // C
    return jnp.zeros((F, H), jnp.float32).at[rows, cols].set(a.reshape(-1))


def _layer(xfeat, srcp, dstp, aeT, aeloop, M, W=None, hmat=None, as_=None,
           ad_=None, H=4, F=256):
    G = F // 32
    mvecs = jnp.broadcast_to(M.reshape(H, 1), (H, 16))
    asT = jnp.asarray(as_.T)
    adT = jnp.asarray(ad_.T)
    esc = _get_pass_a(H)(asT, adT, aeT, srcp, dstp, mvecs)

    hgs = [jnp.asarray(hmat[:, 32 * g:32 * (g + 1)]) for g in range(G)]
    zrows = jnp.zeros((STR, 32), jnp.float32)
    pout, dout = _get_stage2(G, H, F)(srcp, dstp, esc, zrows, *hgs)
    return pout, dout, esc


def kernel(x, edge_index, edge_attr, W1, a_src1, a_dst1, We1, ae1, b1, gnw1,
           gnb1, gnm1, W2, a_src2, a_dst2, We2, ae2, b2, gnw2, gnb2, gnm2,
           Wc, bc):
    f32 = jnp.float32
    xp = jnp.pad(x.astype(f32), ((0, NPAD - N), (0, 0)))
    srcp = jnp.pad(edge_index[0], (0, EP - E))
    dstp = jnp.pad(edge_index[1], (0, EP - E))

    Ps1 = _head_proj(a_src1, 256)
    Pd1 = _head_proj(a_dst1, 256)
    Pae1 = _head_proj(ae1, 256)
    h1, as1, ad1, mxs1, mxd1 = _dense_head(xp, W1, Ps1, Pd1, 4)
    aed1, _sum1, mxe1, aeloop1 = _edge_logits(edge_attr, We1, Pae1, 4)
    M1 = jnp.maximum(mxs1 + mxd1 + mxe1, 0.0)
    aeT1 = jnp.pad(jnp.asarray(aed1.T), ((0, 0), (0, EP - E)))
    p1a, d1a, _ = _layer(xp, srcp, dstp, aeT1, aeloop1, M1.reshape(-1),
                         hmat=h1, as_=as1, ad_=ad1, H=4, F=256)
    Q1 = jnp.asarray(np.repeat(np.eye(4, dtype=np.float32), 64, axis=1))
    op1, cs1, cs21 = _epilogue(p1a[0], p1a[1], d1a[0], d1a[1], h1, as1, ad1,
                               aeloop1, M1, Q1, b1.reshape(1, -1), 4, 256)

    Ps2 = _head_proj(a_src2, 128)
    Pd2 = _head_proj(a_dst2, 128)
    Pae2 = _head_proj(ae2, 128)
    h2, as2, ad2, mxs2, mxd2 = _transform(
        op1, cs1, cs21, gnw1.reshape(1, -1), gnb1.reshape(1, -1),
        gnm1.reshape(1, -1), W2, Ps2, Pd2, 2)
    aed2, _sum2, mxe2, aeloop2 = _edge_logits(edge_attr, We2, Pae2, 2)
    M2 = jnp.maximum(mxs2 + mxd2 + mxe2, 0.0)
    aeT2 = jnp.pad(jnp.asarray(aed2.T), ((0, 0), (0, EP - E)))
    p2a, d2a, _ = _layer(xp, srcp, dstp, aeT2, aeloop2, M2.reshape(-1),
                         hmat=h2, as_=as2, ad_=ad2, H=2, F=128)
    Q2 = jnp.asarray(np.repeat(np.eye(2, dtype=np.float32), 64, axis=1))
    op2, cs2, cs22 = _epilogue(p2a[0], p2a[1], d2a[0], d2a[1], h2, as2, ad2,
                               aeloop2, M2, Q2, b2.reshape(1, -1), 2, 128)

    out = _classifier(op2, cs2, cs22, gnw2.reshape(1, -1), gnb2.reshape(1, -1),
                      gnm2.reshape(1, -1), Wc, bc.reshape(1, -1))
    return out[:N]


# speedup vs baseline: 13.1417x; 1.3069x over previous
import functools

import numpy as np
import jax
import jax.numpy as jnp
from jax import lax
from jax.experimental import pallas as pl
from jax.experimental.pallas import tpu as pltpu
from jax.experimental.pallas import tpu_sc as plsc

N = 50000
E = 800000
NPAD = 50176
EP = 802816
NTILE = 32
STR = NPAD // 16
BLK = 3136
NBLK = NPAD // BLK


def _dense_head_body(x_ref, w_ref, ps_ref, pd_ref,
                     h_ref, as_ref, ad_ref, ms_ref, md_ref):
    i = pl.program_id(0)
    hb = jnp.dot(x_ref[...].astype(jnp.bfloat16), w_ref[...].astype(jnp.bfloat16),
                 preferred_element_type=jnp.float32)
    h_ref[...] = hb
    asb = jnp.dot(hb, ps_ref[...], preferred_element_type=jnp.float32,
                  precision=lax.Precision.HIGHEST)
    adb = jnp.dot(hb, pd_ref[...], preferred_element_type=jnp.float32,
                  precision=lax.Precision.HIGHEST)
    as_ref[...] = asb
    ad_ref[...] = adb

    @pl.when(i == 0)
    def _():
        ms_ref[...] = jnp.full_like(ms_ref, -1e30)
        md_ref[...] = jnp.full_like(md_ref, -1e30)

    ms_ref[...] = jnp.maximum(ms_ref[...], jnp.max(asb, axis=0, keepdims=True))
    md_ref[...] = jnp.maximum(md_ref[...], jnp.max(adb, axis=0, keepdims=True))


def _dense_head(xp, W, Psrc, Pdst, H):
    fin = xp.shape[1]
    fout = W.shape[1]
    return pl.pallas_call(
        _dense_head_body,
        grid=(NBLK,),
        in_specs=[
            pl.BlockSpec((BLK, fin), lambda i: (i, 0)),
            pl.BlockSpec((fin, fout), lambda i: (0, 0)),
            pl.BlockSpec((fout, H), lambda i: (0, 0)),
            pl.BlockSpec((fout, H), lambda i: (0, 0)),
        ],
        out_specs=[
            pl.BlockSpec((BLK, fout), lambda i: (i, 0)),
            pl.BlockSpec((BLK, H), lambda i: (i, 0)),
            pl.BlockSpec((BLK, H), lambda i: (i, 0)),
            pl.BlockSpec((1, H), lambda i: (0, 0)),
            pl.BlockSpec((1, H), lambda i: (0, 0)),
        ],
        out_shape=[
            jax.ShapeDtypeStruct((NPAD, fout), jnp.float32),
            jax.ShapeDtypeStruct((NPAD, H), jnp.float32),
            jax.ShapeDtypeStruct((NPAD, H), jnp.float32),
            jax.ShapeDtypeStruct((1, H), jnp.float32),
            jax.ShapeDtypeStruct((1, H), jnp.float32),
        ],
    )(xp, W, Psrc, Pdst)


def _edge_logits_body(ea_ref, we_ref, pae_ref,
                      aed_ref, sum_ref, mx_ref, loop_ref, *, nblocks):
    i = pl.program_id(0)
    we16 = we_ref[...].astype(jnp.bfloat16).astype(jnp.float32)
    weeff = jnp.dot(we16, pae_ref[...], preferred_element_type=jnp.float32,
                    precision=lax.Precision.HIGHEST)
    eab = ea_ref[...]
    eab16 = eab.astype(jnp.bfloat16).astype(jnp.float32)
    aeb = jnp.dot(eab16, weeff, preferred_element_type=jnp.float32,
                  precision=lax.Precision.HIGHEST)
    aed_ref[...] = aeb

    @pl.when(i == 0)
    def _():
        sum_ref[...] = jnp.zeros_like(sum_ref)
        mx_ref[...] = jnp.full_like(mx_ref, -1e30)

    sum_ref[...] += jnp.sum(eab, axis=0, keepdims=True)
    mx_ref[...] = jnp.maximum(mx_ref[...], jnp.max(aeb, axis=0, keepdims=True))

    @pl.when(i == nblocks - 1)
    def _():
        mean16 = (sum_ref[...] / E).astype(jnp.bfloat16).astype(jnp.float32)
        loop_ref[...] = jnp.dot(mean16, weeff,
                                preferred_element_type=jnp.float32,
                                precision=lax.Precision.HIGHEST)


def _edge_logits(edge_attr, We, Pae, H):
    fe = edge_attr.shape[1]
    blke = 8000
    nblocks = E // blke
    return pl.pallas_call(
        functools.partial(_edge_logits_body, nblocks=nblocks),
        grid=(nblocks,),
        in_specs=[
            pl.BlockSpec((blke, fe), lambda i: (i, 0)),
            pl.BlockSpec((fe, H * 64), lambda i: (0, 0)),
            pl.BlockSpec((H * 64, H), lambda i: (0, 0)),
        ],
        out_specs=[
            pl.BlockSpec((blke, H), lambda i: (i, 0)),
            pl.BlockSpec((1, fe), lambda i: (0, 0)),
            pl.BlockSpec((1, H), lambda i: (0, 0)),
            pl.BlockSpec((1, H), lambda i: (0, 0)),
        ],
        out_shape=[
            jax.ShapeDtypeStruct((E, H), jnp.float32),
            jax.ShapeDtypeStruct((1, fe), jnp.float32),
            jax.ShapeDtypeStruct((1, H), jnp.float32),
            jax.ShapeDtypeStruct((1, H), jnp.float32),
        ],
    )(edge_attr, We, Pae)


def _epilogue_body(p0_ref, p1_ref, d0_ref, d1_ref, h_ref, as_ref, ad_ref,
                   loop_ref, m_ref, q_ref, b_ref,
                   out_ref, cs_ref, cs2_ref, *, H, blk):
    i = pl.program_id(0)
    av = as_ref[...] + ad_ref[...] + loop_ref[...]
    av = jnp.where(av > 0, av, av * 0.2)
    el = jnp.exp(av - m_ref[...])
    den = d0_ref[...][:, :H] + d1_ref[...][:, :H] + el
    el_b = jnp.dot(el, q_ref[...], preferred_element_type=jnp.float32,
                   precision=lax.Precision.HIGHEST)
    den_b = jnp.dot(den, q_ref[...], preferred_element_type=jnp.float32,
                    precision=lax.Precision.HIGHEST)
    num = p0_ref[...] + p1_ref[...] + el_b * h_ref[...]
    op = num / den_b + b_ref[...]
    rid = lax.broadcasted_iota(jnp.int32, (blk, 1), 0) + i * blk
    op = jnp.where(rid < N, op, 0.0)
    out_ref[...] = op

    @pl.when(i == 0)
    def _():
        cs_ref[...] = jnp.zeros_like(cs_ref)
        cs2_ref[...] = jnp.zeros_like(cs2_ref)

    cs_ref[...] += jnp.sum(op, axis=0, keepdims=True)
    cs2_ref[...] += jnp.sum(op * op, axis=0, keepdims=True)


def _epilogue(p0, p1, d0, d1, hmat, as_, ad_, aeloop, M, Q, bvec, H, F):
    return pl.pallas_call(
        functools.partial(_epilogue_body, H=H, blk=BLK),
        grid=(NBLK,),
        in_specs=[
            pl.BlockSpec((BLK, F), lambda i: (i, 0)),
            pl.BlockSpec((BLK, F), lambda i: (i, 0)),
            pl.BlockSpec((BLK, 32), lambda i: (i, 0)),
            pl.BlockSpec((BLK, 32), lambda i: (i, 0)),
            pl.BlockSpec((BLK, F), lambda i: (i, 0)),
            pl.BlockSpec((BLK, H), lambda i: (i, 0)),
            pl.BlockSpec((BLK, H), lambda i: (i, 0)),
            pl.BlockSpec((1, H), lambda i: (0, 0)),
            pl.BlockSpec((1, H), lambda i: (0, 0)),
            pl.BlockSpec((H, F), lambda i: (0, 0)),
            pl.BlockSpec((1, F), lambda i: (0, 0)),
        ],
        out_specs=[
            pl.BlockSpec((BLK, F), lambda i: (i, 0)),
            pl.BlockSpec((1, F), lambda i: (0, 0)),
            pl.BlockSpec((1, F), lambda i: (0, 0)),
        ],
        out_shape=[
            jax.ShapeDtypeStruct((NPAD, F), jnp.float32),
            jax.ShapeDtypeStruct((1, F), jnp.float32),
            jax.ShapeDtypeStruct((1, F), jnp.float32),
        ],
    )(p0, p1, d0, d1, hmat, as_, ad_, aeloop, M, Q, bvec)


def _transform_body(op_ref, cs_ref, cs2_ref, gw_ref, gb_ref, gm_ref,
                    w_ref, ps_ref, pd_ref,
                    h_ref, as_ref, ad_ref, ms_ref, md_ref):
    i = pl.program_id(0)
    mean = cs_ref[...] / N
    ex2 = cs2_ref[...] / N
    msv = gm_ref[...]
    var = ex2 - msv * (2.0 - msv) * mean * mean
    scale = gw_ref[...] / jnp.sqrt(var + 1e-5)
    xb = (op_ref[...] - msv * mean) * scale + gb_ref[...]
    xb = jnp.where(xb > 0, xb, jnp.exp(xb) - 1.0)
    hb = jnp.dot(xb.astype(jnp.bfloat16), w_ref[...].astype(jnp.bfloat16),
                 preferred_element_type=jnp.float32)
    h_ref[...] = hb
    asb = jnp.dot(hb, ps_ref[...], preferred_element_type=jnp.float32,
                  precision=lax.Precision.HIGHEST)
    adb = jnp.dot(hb, pd_ref[...], preferred_element_type=jnp.float32,
                  precision=lax.Precision.HIGHEST)
    as_ref[...] = asb
    ad_ref[...] = adb

    @pl.when(i == 0)
    def _():
        ms_ref[...] = jnp.full_like(ms_ref, -1e30)
        md_ref[...] = jnp.full_like(md_ref, -1e30)

    ms_ref[...] = jnp.maximum(ms_ref[...], jnp.max(asb, axis=0, keepdims=True))
    md_ref[...] = jnp.maximum(md_ref[...], jnp.max(adb, axis=0, keepdims=True))


def _transform(op, cs, cs2, gnw, gnb, gnm, W, Psrc, Pdst, H2):
    F = op.shape[1]
    F2 = W.shape[1]
    return pl.pallas_call(
        _transform_body,
        grid=(NBLK,),
        in_specs=[
            pl.BlockSpec((BLK, F), lambda i: (i, 0)),
            pl.BlockSpec((1, F), lambda i: (0, 0)),
            pl.BlockSpec((1, F), lambda i: (0, 0)),
            pl.BlockSpec((1, F), lambda i: (0, 0)),
            pl.BlockSpec((1, F), lambda i: (0, 0)),
            pl.BlockSpec((1, F), lambda i: (0, 0)),
            pl.BlockSpec((F, F2), lambda i: (0, 0)),
            pl.BlockSpec((F2, H2), lambda i: (0, 0)),
            pl.BlockSpec((F2, H2), lambda i: (0, 0)),
        ],
        out_specs=[
            pl.BlockSpec((BLK, F2), lambda i: (i, 0)),
            pl.BlockSpec((BLK, H2), lambda i: (i, 0)),
            pl.BlockSpec((BLK, H2), lambda i: (i, 0)),
            pl.BlockSpec((1, H2), lambda i: (0, 0)),
            pl.BlockSpec((1, H2), lambda i: (0, 0)),
        ],
        out_shape=[
            jax.ShapeDtypeStruct((NPAD, F2), jnp.float32),
            jax.ShapeDtypeStruct((NPAD, H2), jnp.float32),
            jax.ShapeDtypeStruct((NPAD, H2), jnp.float32),
            jax.ShapeDtypeStruct((1, H2), jnp.float32),
            jax.ShapeDtypeStruct((1, H2), jnp.float32),
        ],
    )(op, cs, cs2, gnw, gnb, gnm, W, Psrc, Pdst)


def _classifier_body(op_ref, cs_ref, cs2_ref, gw_ref, gb_ref, gm_ref,
                     wc_ref, bc_ref, out_ref):
    mean = cs_ref[...] / N
    ex2 = cs2_ref[...] / N
    msv = gm_ref[...]
    var = ex2 - msv * (2.0 - msv) * mean * mean
    scale = gw_ref[...] / jnp.sqrt(var + 1e-5)
    xb = (op_ref[...] - msv * mean) * scale + gb_ref[...]
    xb = jnp.where(xb > 0, xb, jnp.exp(xb) - 1.0)
    out_ref[...] = jnp.dot(xb.astype(jnp.bfloat16), wc_ref[...].astype(jnp.bfloat16),
                           preferred_element_type=jnp.float32) + bc_ref[...]


def _classifier(op, cs, cs2, gnw, gnb, gnm, Wc, bc):
    F = op.shape[1]
    ncls = Wc.shape[1]
    return pl.pallas_call(
        _classifier_body,
        grid=(NBLK,),
        in_specs=[
            pl.BlockSpec((BLK, F), lambda i: (i, 0)),
            pl.BlockSpec((1, F), lambda i: (0, 0)),
            pl.BlockSpec((1, F), lambda i: (0, 0)),
            pl.BlockSpec((1, F), lambda i: (0, 0)),
            pl.BlockSpec((1, F), lambda i: (0, 0)),
            pl.BlockSpec((1, F), lambda i: (0, 0)),
            pl.BlockSpec((F, ncls), lambda i: (0, 0)),
            pl.BlockSpec((1, ncls), lambda i: (0, 0)),
        ],
        out_specs=pl.BlockSpec((BLK, ncls), lambda i: (i, 0)),
        out_shape=jax.ShapeDtypeStruct((NPAD, ncls), jnp.float32),
    )(op, cs, cs2, gnw, gnb, gnm, Wc, bc)


_MESH = dict(core_axis_name="c", subcore_axis_name="s")


def _make_pass_a(H):
    ept = EP // NTILE
    nb = ept // 512

    @functools.partial(
        pl.kernel,
        out_type=jax.ShapeDtypeStruct((H, EP), jnp.float32),
        mesh=plsc.VectorSubcoreMesh(**_MESH),
        compiler_params=pltpu.CompilerParams(needs_layout_passes=False, use_tc_tiling_on_sc=False),
        scratch_types=[
            pltpu.VMEM((NPAD,), jnp.float32),
            pltpu.VMEM((NPAD,), jnp.float32),
            pltpu.VMEM((512,), jnp.int32),
            pltpu.VMEM((512,), jnp.int32),
            pltpu.VMEM((512,), jnp.float32),
            pltpu.VMEM((512,), jnp.float32),
            pltpu.VMEM((16,), jnp.float32),
        ],
    )
    def k(asT, adT, aeT, src, dst, mvecs, esc_out, tA, tB, sbuf, dbuf, aebuf, ebuf, mv):
        c = lax.axis_index("c")
        s = lax.axis_index("s")
        base = (c * 16 + s) * ept
        for h in range(H):
            pltpu.sync_copy(asT.at[h], tA)
            pltpu.sync_copy(adT.at[h], tB)
            pltpu.sync_copy(mvecs.at[h], mv)
            mvv = mv[...]

            def body(b, carry):
                off = base + b * 512
                pltpu.sync_copy(src.at[pl.ds(off, 512)], sbuf)
                pltpu.sync_copy(dst.at[pl.ds(off, 512)], dbuf)
                pltpu.sync_copy(aeT.at[h, pl.ds(off, 512)], aebuf)

                def sub(j, carry2):
                    si = sbuf[pl.ds(j * 16, 16)]
                    di = dbuf[pl.ds(j * 16, 16)]
                    a = (plsc.load_gather(tA, [si])
                         + plsc.load_gather(tB, [di])
                         + aebuf[pl.ds(j * 16, 16)])
                    a = jnp.where(a > 0, a, a * 0.2)
                    ev = jnp.exp(a - mvv)
                    gidx = off + j * 16 + lax.iota(jnp.int32, 16)
                    ev = jnp.where(gidx < E, ev, 0.0)
                    ebuf[pl.ds(j * 16, 16)] = ev
                    return carry2

                lax.fori_loop(0, 32, sub, 0)
                pltpu.sync_copy(ebuf, esc_out.at[h, pl.ds(off, 512)])
                return carry

            lax.fori_loop(0, nb, body, 0)

    return k


def _make_stage2(G, H, F):
    epc = EP // 2
    ept = epc // 16
    nbt = ept // 128

    @functools.partial(
        pl.kernel,
        out_type=(
            jax.ShapeDtypeStruct((G, 2, NPAD, 32), jnp.float32),
            jax.ShapeDtypeStruct((2, NPAD, 32), jnp.float32),
        ),
        mesh=plsc.VectorSubcoreMesh(**_MESH),
        compiler_params=pltpu.CompilerParams(needs_layout_passes=False, use_tc_tiling_on_sc=False),
        scratch_types=[
            pltpu.VMEM_SHARED((NPAD, 32), jnp.float32),
            pltpu.VMEM((4, 128), jnp.int32),
            pltpu.VMEM((4, 128), jnp.int32),
            pltpu.VMEM((4, 128), jnp.float32),
            pltpu.VMEM((4, 4, 128), jnp.float32),
            pltpu.VMEM((4, 128, 32), jnp.float32),
        ] + [pltpu.SemaphoreType.DMA] * 12,
    )
    def k(src, dst, escF, zrows, hgall, pout, dout, *rest):
        acc, idxb, dstb, escb, escb2, rowb = rest[:6]
        lsem = list(rest[6:10])
        gsem = list(rest[10:14])
        ssem = list(rest[14:18])
        c = lax.axis_index("c")
        s = lax.axis_index("s")
        base = c * epc + s * ept
        roff = s * STR
        iota16 = lax.iota(jnp.int32, 16)

        def fire_loads(m, kb, eoff):
            off = base + m * 128
            pltpu.async_copy(src.at[pl.ds(off, 128)], idxb.at[kb], lsem[kb])
            pltpu.async_copy(dst.at[pl.ds(off, 128)], dstb.at[kb], lsem[kb])
            pltpu.async_copy(escF.at[pl.ds(eoff + off, 128)], escb.at[kb], lsem[kb])

        def wait_loads(m, kb, eoff):
            off = base + m * 128
            pltpu.make_async_copy(src.at[pl.ds(off, 128)], idxb.at[kb], lsem[kb]).wait()
            pltpu.make_async_copy(dst.at[pl.ds(off, 128)], dstb.at[kb], lsem[kb]).wait()
            pltpu.make_async_copy(escF.at[pl.ds(eoff + off, 128)], escb.at[kb],
                                  lsem[kb]).wait()

        def add_goff(kb, goff):
            def go(j, carry2):
                idxb.at[kb][pl.ds(j * 16, 16)] = (
                    idxb.at[kb][pl.ds(j * 16, 16)] + goff)
                return carry2
            lax.fori_loop(0, 8, go, 0)

        def gpass(g, carry):
            goff = g * NPAD
            eoff = (g // 2) * EP
            pltpu.sync_copy(zrows, acc.at[pl.ds(roff, STR)])
            plsc.subcore_barrier()

            fire_loads(0, 0, eoff)
            fire_loads(1, 1, eoff)
            wait_loads(0, 0, eoff)
            add_goff(0, goff)
            pltpu.async_copy(hgall.at[idxb.at[0]], rowb.at[0], gsem[0])

            def outer(bo, carry1):
                for kk in range(4):
                    m = bo * 4 + kk
                    kf = (kk + 2) % 4
                    kn = (kk + 1) % 4

                    @pl.when(m >= 2)
                    def _():
                        pltpu.make_async_copy(rowb.at[kf], acc.at[dstb.at[kf]],
                                              ssem[kf]).wait()

                    @pl.when(m + 2 < nbt)
                    def _():
                        fire_loads(m + 2, kf, eoff)

                    @pl.when(m + 1 < nbt)
                    def _():
                        wait_loads(m + 1, kn, eoff)
                        add_goff(kn, goff)
                        pltpu.async_copy(hgall.at[idxb.at[kn]], rowb.at[kn],
                                         gsem[kn])

                    pltpu.make_async_copy(hgall.at[idxb.at[kk]], rowb.at[kk],
                                          gsem[kk]).wait()

                    def sub(j, carry2, _kk=kk):
                        ridx = j * 16 + iota16
                        ev = escb.at[_kk][pl.ds(j * 16, 16)]
                        for col in range(32):
                            ci = jnp.full((16,), col, jnp.int32)
                            v = plsc.load_gather(rowb.at[_kk], [ridx, ci])
                            plsc.store_scatter(rowb.at[_kk], [ridx, ci], v * ev)
                        return carry2

                    lax.fori_loop(0, 8, sub, 0)
                    pltpu.async_copy(rowb.at[kk], acc.at[dstb.at[kk]], ssem[kk],
                                     add=True)
                return carry1

            lax.fori_loop(0, nbt // 4, outer, 0)
            pltpu.make_async_copy(rowb.at[2], acc.at[dstb.at[2]], ssem[2]).wait()
            pltpu.make_async_copy(rowb.at[3], acc.at[dstb.at[3]], ssem[3]).wait()
            plsc.subcore_barrier()
            pltpu.sync_copy(acc.at[pl.ds(roff, STR)],
                            pout.at[g, c, pl.ds(roff, STR)])
            pltpu.sync_copy(zrows, acc.at[pl.ds(roff, STR)])
            plsc.subcore_barrier()
            return carry

        lax.fori_loop(0, G, gpass, 0)

        for kk in range(4):
            pltpu.sync_copy(zrows.at[pl.ds(0, 128)], rowb.at[kk])

        def dfire(m, kb):
            off = base + m * 128
            pltpu.async_copy(dst.at[pl.ds(off, 128)], dstb.at[kb], lsem[kb])
            for h in range(H):
                pltpu.async_copy(escF.at[pl.ds(h * EP + off, 128)],
                                 escb2.at[kb, h], lsem[kb])

        def dwait(m, kb):
            off = base + m * 128
            pltpu.make_async_copy(dst.at[pl.ds(off, 128)], dstb.at[kb], lsem[kb]).wait()
            for h in range(H):
                pltpu.make_async_copy(escF.at[pl.ds(h * EP + off, 128)],
                                      escb2.at[kb, h], lsem[kb]).wait()

        dfire(0, 0)
        dfire(1, 1)

        def douter(bo, carry):
            for kk in range(4):
                m = bo * 4 + kk
                kf = (kk + 2) % 4

                @pl.when(m >= 2)
                def _():
                    pltpu.make_async_copy(rowb.at[kf], acc.at[dstb.at[kf]],
                                          ssem[kf]).wait()

                @pl.when(m + 2 < nbt)
                def _():
                    dfire(m + 2, kf)

                dwait(m, kk)

                for h in range(H):
                    def dsub(j, carry2, _h=h, _kk=kk):
                        ridx = j * 16 + iota16
                        ev = escb2.at[_kk, _h][pl.ds(j * 16, 16)]
                        ci = jnp.full((16,), _h, jnp.int32)
                        plsc.store_scatter(rowb.at[_kk], [ridx, ci], ev)
                        return carry2

                    lax.fori_loop(0, 8, dsub, 0)
                pltpu.async_copy(rowb.at[kk], acc.at[dstb.at[kk]], ssem[kk],
                                 add=True)
            return carry

        lax.fori_loop(0, nbt // 4, douter, 0)
        pltpu.make_async_copy(rowb.at[2], acc.at[dstb.at[2]], ssem[2]).wait()
        pltpu.make_async_copy(rowb.at[3], acc.at[dstb.at[3]], ssem[3]).wait()
        plsc.subcore_barrier()
        pltpu.sync_copy(acc.at[pl.ds(roff, STR)], dout.at[c, pl.ds(roff, STR)])

    return k


_PASS_A = {}
_STAGE2 = {}


def _get_pass_a(H):
    if H not in _PASS_A:
        _PASS_A[H] = _make_pass_a(H)
    return _PASS_A[H]


def _get_stage2(G, H, F):
    key = (G, H, F)
    if key not in _STAGE2:
        _STAGE2[key] = _make_stage2(G, H, F)
    return _STAGE2[key]


def _head_proj(a, F):
    H, C = a.shape
    rows = jnp.arange(F)
    cols = rows // C
    return jnp.zeros((F, H), jnp.float32).at[rows, cols].set(a.reshape(-1))


def _layer(xfeat, srcp, dstp, aeT, aeloop, M, W=None, hmat=None, as_=None,
           ad_=None, H=4, F=256):
    G = F // 32
    mvecs = jnp.broadcast_to(M.reshape(H, 1), (H, 16))
    asT = jnp.asarray(as_.T)
    adT = jnp.asarray(ad_.T)
    esc = _get_pass_a(H)(asT, adT, aeT, srcp, dstp, mvecs)

    hgall = jnp.transpose(hmat.reshape(NPAD, G, 32), (1, 0, 2)).reshape(
        G * NPAD, 32)
    zrows = jnp.zeros((STR, 32), jnp.float32)
    pg, dout = _get_stage2(G, H, F)(srcp, dstp, esc.reshape(-1), zrows, hgall)
    pout = jnp.transpose(pg, (1, 2, 0, 3)).reshape(2, NPAD, F)
    return pout, dout, esc


def kernel(x, edge_index, edge_attr, W1, a_src1, a_dst1, We1, ae1, b1, gnw1,
           gnb1, gnm1, W2, a_src2, a_dst2, We2, ae2, b2, gnw2, gnb2, gnm2,
           Wc, bc):
    f32 = jnp.float32
    xp = jnp.pad(x.astype(f32), ((0, NPAD - N), (0, 0)))
    srcp = jnp.pad(edge_index[0], (0, EP - E))
    dstp = jnp.pad(edge_index[1], (0, EP - E))

    Ps1 = _head_proj(a_src1, 256)
    Pd1 = _head_proj(a_dst1, 256)
    Pae1 = _head_proj(ae1, 256)
    h1, as1, ad1, mxs1, mxd1 = _dense_head(xp, W1, Ps1, Pd1, 4)
    aed1, _sum1, mxe1, aeloop1 = _edge_logits(edge_attr, We1, Pae1, 4)
    M1 = jnp.maximum(mxs1 + mxd1 + mxe1, 0.0)
    aeT1 = jnp.pad(jnp.asarray(aed1.T), ((0, 0), (0, EP - E)))
    p1a, d1a, _ = _layer(xp, srcp, dstp, aeT1, aeloop1, M1.reshape(-1),
                         hmat=h1, as_=as1, ad_=ad1, H=4, F=256)
    Q1 = jnp.asarray(np.repeat(np.eye(4, dtype=np.float32), 64, axis=1))
    op1, cs1, cs21 = _epilogue(p1a[0], p1a[1], d1a[0], d1a[1], h1, as1, ad1,
                               aeloop1, M1, Q1, b1.reshape(1, -1), 4, 256)

    Ps2 = _head_proj(a_src2, 128)
    Pd2 = _head_proj(a_dst2, 128)
    Pae2 = _head_proj(ae2, 128)
    h2, as2, ad2, mxs2, mxd2 = _transform(
        op1, cs1, cs21, gnw1.reshape(1, -1), gnb1.reshape(1, -1),
        gnm1.reshape(1, -1), W2, Ps2, Pd2, 2)
    aed2, _sum2, mxe2, aeloop2 = _edge_logits(edge_attr, We2, Pae2, 2)
    M2 = jnp.maximum(mxs2 + mxd2 + mxe2, 0.0)
    aeT2 = jnp.pad(jnp.asarray(aed2.T), ((0, 0), (0, EP - E)))
    p2a, d2a, _ = _layer(xp, srcp, dstp, aeT2, aeloop2, M2.reshape(-1),
                         hmat=h2, as_=as2, ad_=ad2, H=2, F=128)
    Q2 = jnp.asarray(np.repeat(np.eye(2, dtype=np.float32), 64, axis=1))
    op2, cs2, cs22 = _epilogue(p2a[0], p2a[1], d2a[0], d2a[1], h2, as2, ad2,
                               aeloop2, M2, Q2, b2.reshape(1, -1), 2, 128)

    out = _classifier(op2, cs2, cs22, gnw2.reshape(1, -1), gnb2.reshape(1, -1),
                      gnm2.reshape(1, -1), Wc, bc.reshape(1, -1))
    return out[:N]
